# speedup vs baseline: 3.8390x; 3.8390x over previous
import functools

import jax
import jax.numpy as jnp
from jax import lax
from jax.experimental import pallas as pl
from jax.experimental.pallas import tpu as pltpu
from jax.experimental.pallas import tpu_sc as plsc

VOCAB = 100000
EMB_DIM = 768
BATCH = 16384


def _sc_gather(table, idx):
    info = plsc.get_sparse_core_info()
    nw = info.num_cores * info.num_subcores
    b_per_w = BATCH // nw
    CH = 128
    n_ch = b_per_w // CH

    mesh = plsc.VectorSubcoreMesh(core_axis_name="c", subcore_axis_name="s")

    @functools.partial(
        pl.kernel,
        mesh=mesh,
        out_type=jax.ShapeDtypeStruct((BATCH, EMB_DIM), jnp.float32),
        scratch_types=[
            pltpu.VMEM((b_per_w,), jnp.int32),
            pltpu.VMEM((CH, EMB_DIM), jnp.float32),
            pltpu.SemaphoreType.DMA,
        ],
    )
    def gather_kernel(table_hbm, idx_hbm, out_hbm, idx_v, rows_v, sem):
        wid = lax.axis_index("s") * info.num_cores + lax.axis_index("c")
        base = wid * b_per_w
        pltpu.sync_copy(idx_hbm.at[pl.ds(base, b_per_w)], idx_v)

        def body(c, _):
            off = c * CH
            pltpu.async_copy(
                table_hbm.at[idx_v.at[pl.ds(off, CH)]], rows_v, sem
            ).wait()
            pltpu.sync_copy(rows_v, out_hbm.at[pl.ds(base + off, CH)])
            return 0

        lax.fori_loop(0, n_ch, body, 0)

    return gather_kernel(table, idx)


_BM = 1024


def _mm_body(x_ref, w_ref, tt_ref, wl_ref, b_ref, o_ref):
    acc = lax.dot_general(
        x_ref[...], w_ref[...],
        (((1,), (1,)), ((), ())),
        preferred_element_type=jnp.float32,
    )
    o_ref[...] = acc + tt_ref[...] * wl_ref[...] + b_ref[...]


def _tc_linear(emb, tt, w1, wlast, bias):
    grid = (BATCH // _BM,)
    return pl.pallas_call(
        _mm_body,
        grid=grid,
        in_specs=[
            pl.BlockSpec((_BM, EMB_DIM), lambda i: (i, 0)),
            pl.BlockSpec((EMB_DIM, EMB_DIM), lambda i: (0, 0)),
            pl.BlockSpec((_BM, 1), lambda i: (i, 0)),
            pl.BlockSpec((1, EMB_DIM), lambda i: (0, 0)),
            pl.BlockSpec((1, EMB_DIM), lambda i: (0, 0)),
        ],
        out_specs=pl.BlockSpec((_BM, EMB_DIM), lambda i: (i, 0)),
        out_shape=jax.ShapeDtypeStruct((BATCH, EMB_DIM), jnp.float32),
    )(emb, w1, tt, wlast, bias)


def kernel(qubit, total_time, emb_table, W, b):
    idx = qubit.astype(jnp.int32)
    emb = _sc_gather(emb_table, idx)
    w1 = W[:, :EMB_DIM]
    wlast = W[:, EMB_DIM].reshape(1, EMB_DIM)
    bias = b.reshape(1, EMB_DIM)
    return _tc_linear(emb, total_time, w1, wlast, bias)
